# Initial kernel scaffold; baseline (speedup 1.0000x reference)
import jax
import jax.numpy as jnp
from jax.experimental import pallas as pl


def kernel(x, edge_index, batch, W1, b1, W2, b2, W3, b3, W4, b4, Wh, bh):
    raise NotImplementedError("write your pallas kernel here")


# speedup vs baseline: 15.3855x; 15.3855x over previous
import functools

import jax
import jax.numpy as jnp
from jax import lax
from jax.experimental import pallas as pl
from jax.experimental.pallas import tpu as pltpu
from jax.experimental.pallas import tpu_sc as plsc

N = 10000
E = 320000
F = 128
G = 64
NC = 2
NS = 16
L = 16
NW = NC * NS
EPW = E // NW
K = 40
NCHUNK = EPW // K
NBUF = 2
NPT = N // NS
BN = 2000
GRID = N // BN


def _sc_scatter_body(y_hbm, row_hbm, col_hbm, z_hbm, out_hbm,
                     acc, ridx, cidx0, cidx1, rows, g0, g1, i0, i1):
    c = lax.axis_index("c")
    s = lax.axis_index("s")
    wid = s * NC + c
    cidx = (cidx0, cidx1)
    gsem = (g0, g1)
    isem = (i0, i1)

    pltpu.sync_copy(row_hbm.at[wid], ridx)
    pltpu.sync_copy(z_hbm.at[s], acc.at[pl.ds(s * NPT, NPT)])

    for b in range(NBUF):
        pltpu.async_copy(col_hbm.at[wid, b, 0], cidx[b], isem[b])
        pltpu.async_copy(y_hbm.at[ridx.at[b]], rows.at[b], gsem[b])
    plsc.subcore_barrier()

    def _wait_g(b):
        pltpu.make_async_copy(y_hbm.at[ridx.at[0]], rows.at[b], gsem[b]).wait()

    def _wait_i(b):
        pltpu.make_async_copy(col_hbm.at[wid, 0, 0], cidx[b], isem[b]).wait()

    def _group(g, carry):
        for b in range(NBUF):
            j = g * NBUF + b
            _wait_g(b)
            _wait_i(b)
            pltpu.sync_copy(rows.at[b], acc.at[cidx[b]], add=True)
            pltpu.async_copy(col_hbm.at[wid, j + NBUF, 0], cidx[b], isem[b])
            pltpu.async_copy(y_hbm.at[ridx.at[j + NBUF]], rows.at[b], gsem[b])
        return carry

    lax.fori_loop(0, (NCHUNK - NBUF) // NBUF, _group, 0)
    for b in range(NBUF):
        _wait_g(b)
        _wait_i(b)
        pltpu.sync_copy(rows.at[b], acc.at[cidx[b]], add=True)
    plsc.subcore_barrier()

    pltpu.sync_copy(acc.at[pl.ds(s * NPT, NPT)], out_hbm.at[c * NS + s])


@functools.lru_cache(maxsize=None)
def _sc_kernels():
    mesh = plsc.VectorSubcoreMesh(core_axis_name="c", subcore_axis_name="s",
                                  num_cores=NC, num_subcores=NS)
    scatter = functools.partial(
        pl.kernel,
        out_type=jax.ShapeDtypeStruct((NC * NS, NPT, F), jnp.float32),
        mesh=mesh,
        scratch_types=[
            pltpu.VMEM_SHARED((N, F), jnp.float32),
            pltpu.VMEM((NCHUNK, K), jnp.int32),
            pltpu.VMEM((K,), jnp.int32),
            pltpu.VMEM((K,), jnp.int32),
            pltpu.VMEM((NBUF, K, F), jnp.float32),
            pltpu.SemaphoreType.DMA,
            pltpu.SemaphoreType.DMA,
            pltpu.SemaphoreType.DMA,
            pltpu.SemaphoreType.DMA,
        ],
    )(_sc_scatter_body)
    return scatter


def _tc_pre_body(p_ref, x_ref, w_ref, y_ref, dis_ref):
    deg = p_ref[0, :, 0:1] + p_ref[1, :, 0:1] + 1.0
    dis = lax.rsqrt(deg)
    dis_ref[...] = dis
    y_ref[...] = dis * jnp.dot(x_ref[...], w_ref[...],
                               preferred_element_type=jnp.float32)


def _tc_pre(p, x, w):
    return pl.pallas_call(
        _tc_pre_body,
        grid=(GRID,),
        in_specs=[
            pl.BlockSpec((NC, BN, F), lambda i: (0, i, 0)),
            pl.BlockSpec((BN, F), lambda i: (i, 0)),
            pl.BlockSpec((F, F), lambda i: (0, 0)),
        ],
        out_specs=[
            pl.BlockSpec((BN, F), lambda i: (i, 0)),
            pl.BlockSpec((BN, 1), lambda i: (i, 0)),
        ],
        out_shape=[
            jax.ShapeDtypeStruct((N, F), jnp.float32),
            jax.ShapeDtypeStruct((N, 1), jnp.float32),
        ],
    )(p, x, w)


def _tc_mid_body(s_ref, y_ref, dis_ref, b_ref, w_ref, yo_ref):
    t = s_ref[0] + s_ref[1] + y_ref[...]
    h = jnp.maximum(dis_ref[...] * t + b_ref[...], 0.0)
    yo_ref[...] = dis_ref[...] * jnp.dot(h, w_ref[...],
                                         preferred_element_type=jnp.float32)


def _tc_mid(s, y, dis, b, w):
    return pl.pallas_call(
        _tc_mid_body,
        grid=(GRID,),
        in_specs=[
            pl.BlockSpec((NC, BN, F), lambda i: (0, i, 0)),
            pl.BlockSpec((BN, F), lambda i: (i, 0)),
            pl.BlockSpec((BN, 1), lambda i: (i, 0)),
            pl.BlockSpec((1, F), lambda i: (0, 0)),
            pl.BlockSpec((F, F), lambda i: (0, 0)),
        ],
        out_specs=pl.BlockSpec((BN, F), lambda i: (i, 0)),
        out_shape=jax.ShapeDtypeStruct((N, F), jnp.float32),
    )(s, y, dis, b, w)


def _tc_fin_body(s_ref, y_ref, dis_ref, b_ref, seg_ref, wh_ref, bh_ref,
                 out_ref, sums, cnts):
    i = pl.program_id(0)

    @pl.when(i == 0)
    def _init():
        sums[...] = jnp.zeros_like(sums)
        cnts[...] = jnp.zeros_like(cnts)

    t = s_ref[0] + s_ref[1] + y_ref[...]
    h = dis_ref[...] * t + b_ref[...]
    onehot = (seg_ref[...] == lax.broadcasted_iota(jnp.int32, (1, G), 1)
              ).astype(jnp.float32)
    dn = (((0,), (0,)), ((), ()))
    sums[...] += lax.dot_general(onehot, h, dn,
                                 preferred_element_type=jnp.float32)
    cnts[...] += lax.dot_general(onehot, jnp.ones_like(h), dn,
                                 preferred_element_type=jnp.float32)

    @pl.when(i == GRID - 1)
    def _fin():
        pooled = sums[...] / jnp.maximum(cnts[...], 1.0)
        out_ref[...] = jnp.dot(pooled, wh_ref[...],
                               preferred_element_type=jnp.float32) + bh_ref[...]


def _tc_fin(s, y, dis, b, seg, wh, bh):
    return pl.pallas_call(
        _tc_fin_body,
        grid=(GRID,),
        in_specs=[
            pl.BlockSpec((NC, BN, F), lambda i: (0, i, 0)),
            pl.BlockSpec((BN, F), lambda i: (i, 0)),
            pl.BlockSpec((BN, 1), lambda i: (i, 0)),
            pl.BlockSpec((1, F), lambda i: (0, 0)),
            pl.BlockSpec((BN, 1), lambda i: (i, 0)),
            pl.BlockSpec((F, 1), lambda i: (0, 0)),
            pl.BlockSpec((1, 1), lambda i: (0, 0)),
        ],
        out_specs=pl.BlockSpec((G, 1), lambda i: (0, 0)),
        out_shape=jax.ShapeDtypeStruct((G, 1), jnp.float32),
        scratch_shapes=[
            pltpu.VMEM((G, F), jnp.float32),
            pltpu.VMEM((G, F), jnp.float32),
        ],
    )(s, y, dis, b, seg, wh, bh)


def kernel(x, edge_index, batch, W1, b1, W2, b2, W3, b3, W4, b4, Wh, bh):
    row = edge_index[0].astype(jnp.int32).reshape(NW, NCHUNK, K)
    col = edge_index[1].astype(jnp.int32).reshape(NW, NCHUNK, 1, K)
    seg = batch.astype(jnp.int32).reshape(N, 1)
    zf = jnp.zeros((NS, NPT, F), jnp.float32)
    ones_nf = jnp.ones((N, F), jnp.float32)
    _sc_scatter = _sc_kernels()

    p = _sc_scatter(ones_nf, row, col, zf).reshape(NC, N, F)
    y, dis = _tc_pre(p, x, W1)

    s = _sc_scatter(y, row, col, zf).reshape(NC, N, F)
    y = _tc_mid(s, y, dis, b1.reshape(1, F), W2)
    s = _sc_scatter(y, row, col, zf).reshape(NC, N, F)
    y = _tc_mid(s, y, dis, b2.reshape(1, F), W3)
    s = _sc_scatter(y, row, col, zf).reshape(NC, N, F)
    y = _tc_mid(s, y, dis, b3.reshape(1, F), W4)
    s = _sc_scatter(y, row, col, zf).reshape(NC, N, F)
    return _tc_fin(s, y, dis, b4.reshape(1, F), seg, Wh, bh.reshape(1, 1))


# speedup vs baseline: 19.8649x; 1.2911x over previous
import functools

import jax
import jax.numpy as jnp
from jax import lax
from jax.experimental import pallas as pl
from jax.experimental.pallas import tpu as pltpu
from jax.experimental.pallas import tpu_sc as plsc

N = 10000
E = 320000
F = 128
G = 64
NC = 2
NS = 16
L = 16
NW = NC * NS
EPW = E // NW
K = 80
NCHUNK = EPW // K
NBUF = 2
NPT = N // NS
BN = 2000
GRID = N // BN


def _sc_scatter_body(y_hbm, row_hbm, col_hbm, z_hbm, out_hbm,
                     acc, ridx, cidx0, cidx1, rows, g0, g1, i0, i1):
    c = lax.axis_index("c")
    s = lax.axis_index("s")
    wid = s * NC + c
    cidx = (cidx0, cidx1)
    gsem = (g0, g1)
    isem = (i0, i1)

    pltpu.sync_copy(row_hbm.at[wid], ridx)
    pltpu.sync_copy(z_hbm.at[s], acc.at[pl.ds(s * NPT, NPT)])

    for b in range(NBUF):
        pltpu.async_copy(col_hbm.at[wid, b, 0], cidx[b], isem[b])
        pltpu.async_copy(y_hbm.at[ridx.at[b]], rows.at[b], gsem[b])
    plsc.subcore_barrier()

    def _wait_g(b):
        pltpu.make_async_copy(y_hbm.at[ridx.at[0]], rows.at[b], gsem[b]).wait()

    def _wait_i(b):
        pltpu.make_async_copy(col_hbm.at[wid, 0, 0], cidx[b], isem[b]).wait()

    def _group(g, carry):
        for b in range(NBUF):
            j = g * NBUF + b
            _wait_g(b)
            _wait_i(b)
            pltpu.sync_copy(rows.at[b], acc.at[cidx[b]], add=True)
            pltpu.async_copy(col_hbm.at[wid, j + NBUF, 0], cidx[b], isem[b])
            pltpu.async_copy(y_hbm.at[ridx.at[j + NBUF]], rows.at[b], gsem[b])
        return carry

    NG = (NCHUNK - NBUF) // NBUF
    lax.fori_loop(0, NG, _group, 0)
    for j in range(NG * NBUF, NCHUNK):
        b = j % NBUF
        _wait_g(b)
        _wait_i(b)
        pltpu.sync_copy(rows.at[b], acc.at[cidx[b]], add=True)
        if j + NBUF < NCHUNK:
            pltpu.async_copy(col_hbm.at[wid, j + NBUF, 0], cidx[b], isem[b])
            pltpu.async_copy(y_hbm.at[ridx.at[j + NBUF]], rows.at[b], gsem[b])
    plsc.subcore_barrier()

    pltpu.sync_copy(acc.at[pl.ds(s * NPT, NPT)], out_hbm.at[c * NS + s])


@functools.lru_cache(maxsize=None)
def _sc_kernels():
    mesh = plsc.VectorSubcoreMesh(core_axis_name="c", subcore_axis_name="s",
                                  num_cores=NC, num_subcores=NS)
    scatter = functools.partial(
        pl.kernel,
        out_type=jax.ShapeDtypeStruct((NC * NS, NPT, F), jnp.float32),
        mesh=mesh,
        scratch_types=[
            pltpu.VMEM_SHARED((N, F), jnp.float32),
            pltpu.VMEM((NCHUNK, K), jnp.int32),
            pltpu.VMEM((K,), jnp.int32),
            pltpu.VMEM((K,), jnp.int32),
            pltpu.VMEM((NBUF, K, F), jnp.float32),
            pltpu.SemaphoreType.DMA,
            pltpu.SemaphoreType.DMA,
            pltpu.SemaphoreType.DMA,
            pltpu.SemaphoreType.DMA,
        ],
    )(_sc_scatter_body)
    return scatter


def _tc_pre_body(p_ref, x_ref, w_ref, y_ref, dis_ref):
    deg = p_ref[0, :, 0:1] + p_ref[1, :, 0:1] + 1.0
    dis = lax.rsqrt(deg)
    dis_ref[...] = dis
    y_ref[...] = dis * jnp.dot(x_ref[...], w_ref[...],
                               preferred_element_type=jnp.float32)


def _tc_pre(p, x, w):
    return pl.pallas_call(
        _tc_pre_body,
        grid=(GRID,),
        in_specs=[
            pl.BlockSpec((NC, BN, F), lambda i: (0, i, 0)),
            pl.BlockSpec((BN, F), lambda i: (i, 0)),
            pl.BlockSpec((F, F), lambda i: (0, 0)),
        ],
        out_specs=[
            pl.BlockSpec((BN, F), lambda i: (i, 0)),
            pl.BlockSpec((BN, 1), lambda i: (i, 0)),
        ],
        out_shape=[
            jax.ShapeDtypeStruct((N, F), jnp.float32),
            jax.ShapeDtypeStruct((N, 1), jnp.float32),
        ],
    )(p, x, w)


def _tc_mid_body(s_ref, y_ref, dis_ref, b_ref, w_ref, yo_ref):
    t = s_ref[0] + s_ref[1] + y_ref[...]
    h = jnp.maximum(dis_ref[...] * t + b_ref[...], 0.0)
    yo_ref[...] = dis_ref[...] * jnp.dot(h, w_ref[...],
                                         preferred_element_type=jnp.float32)


def _tc_mid(s, y, dis, b, w):
    return pl.pallas_call(
        _tc_mid_body,
        grid=(GRID,),
        in_specs=[
            pl.BlockSpec((NC, BN, F), lambda i: (0, i, 0)),
            pl.BlockSpec((BN, F), lambda i: (i, 0)),
            pl.BlockSpec((BN, 1), lambda i: (i, 0)),
            pl.BlockSpec((1, F), lambda i: (0, 0)),
            pl.BlockSpec((F, F), lambda i: (0, 0)),
        ],
        out_specs=pl.BlockSpec((BN, F), lambda i: (i, 0)),
        out_shape=jax.ShapeDtypeStruct((N, F), jnp.float32),
    )(s, y, dis, b, w)


def _tc_fin_body(s_ref, y_ref, dis_ref, b_ref, seg_ref, wh_ref, bh_ref,
                 out_ref, sums, cnts):
    i = pl.program_id(0)

    @pl.when(i == 0)
    def _init():
        sums[...] = jnp.zeros_like(sums)
        cnts[...] = jnp.zeros_like(cnts)

    t = s_ref[0] + s_ref[1] + y_ref[...]
    h = dis_ref[...] * t + b_ref[...]
    onehot = (seg_ref[...] == lax.broadcasted_iota(jnp.int32, (1, G), 1)
              ).astype(jnp.float32)
    dn = (((0,), (0,)), ((), ()))
    sums[...] += lax.dot_general(onehot, h, dn,
                                 preferred_element_type=jnp.float32)
    cnts[...] += lax.dot_general(onehot, jnp.ones_like(h), dn,
                                 preferred_element_type=jnp.float32)

    @pl.when(i == GRID - 1)
    def _fin():
        pooled = sums[...] / jnp.maximum(cnts[...], 1.0)
        out_ref[...] = jnp.dot(pooled, wh_ref[...],
                               preferred_element_type=jnp.float32) + bh_ref[...]


def _tc_fin(s, y, dis, b, seg, wh, bh):
    return pl.pallas_call(
        _tc_fin_body,
        grid=(GRID,),
        in_specs=[
            pl.BlockSpec((NC, BN, F), lambda i: (0, i, 0)),
            pl.BlockSpec((BN, F), lambda i: (i, 0)),
            pl.BlockSpec((BN, 1), lambda i: (i, 0)),
            pl.BlockSpec((1, F), lambda i: (0, 0)),
            pl.BlockSpec((BN, 1), lambda i: (i, 0)),
            pl.BlockSpec((F, 1), lambda i: (0, 0)),
            pl.BlockSpec((1, 1), lambda i: (0, 0)),
        ],
        out_specs=pl.BlockSpec((G, 1), lambda i: (0, 0)),
        out_shape=jax.ShapeDtypeStruct((G, 1), jnp.float32),
        scratch_shapes=[
            pltpu.VMEM((G, F), jnp.float32),
            pltpu.VMEM((G, F), jnp.float32),
        ],
    )(s, y, dis, b, seg, wh, bh)


def kernel(x, edge_index, batch, W1, b1, W2, b2, W3, b3, W4, b4, Wh, bh):
    row = edge_index[0].astype(jnp.int32).reshape(NW, NCHUNK, K)
    col = edge_index[1].astype(jnp.int32).reshape(NW, NCHUNK, 1, K)
    seg = batch.astype(jnp.int32).reshape(N, 1)
    zf = jnp.zeros((NS, NPT, F), jnp.float32)
    ones_nf = jnp.ones((N, F), jnp.float32)
    _sc_scatter = _sc_kernels()

    p = _sc_scatter(ones_nf, row, col, zf).reshape(NC, N, F)
    y, dis = _tc_pre(p, x, W1)

    s = _sc_scatter(y, row, col, zf).reshape(NC, N, F)
    y = _tc_mid(s, y, dis, b1.reshape(1, F), W2)
    s = _sc_scatter(y, row, col, zf).reshape(NC, N, F)
    y = _tc_mid(s, y, dis, b2.reshape(1, F), W3)
    s = _sc_scatter(y, row, col, zf).reshape(NC, N, F)
    y = _tc_mid(s, y, dis, b3.reshape(1, F), W4)
    s = _sc_scatter(y, row, col, zf).reshape(NC, N, F)
    return _tc_fin(s, y, dis, b4.reshape(1, F), seg, Wh, bh.reshape(1, 1))


# speedup vs baseline: 21.1175x; 1.0631x over previous
import functools

import jax
import jax.numpy as jnp
from jax import lax
from jax.experimental import pallas as pl
from jax.experimental.pallas import tpu as pltpu
from jax.experimental.pallas import tpu_sc as plsc

N = 10000
E = 320000
F = 128
G = 64
NC = 2
NS = 16
L = 16
NW = NC * NS
EPW = E // NW
K = 80
NCHUNK = EPW // K
NBUF = 2
NPT = N // NS
BN = 2000
GRID = N // BN


def _sc_scatter_body(y_hbm, row_hbm, col_hbm, z_hbm, out_hbm,
                     acc, ridx, cidx0, cidx1, rows, g0, g1, i0, i1):
    c = lax.axis_index("c")
    s = lax.axis_index("s")
    wid = s * NC + c
    cidx = (cidx0, cidx1)
    gsem = (g0, g1)
    isem = (i0, i1)

    pltpu.sync_copy(row_hbm.at[wid], ridx)
    pltpu.sync_copy(z_hbm.at[s], acc.at[pl.ds(s * NPT, NPT)])

    for b in range(NBUF):
        pltpu.async_copy(col_hbm.at[wid, b, 0], cidx[b], isem[b])
        pltpu.async_copy(y_hbm.at[ridx.at[b]], rows.at[b], gsem[b])
    plsc.subcore_barrier()

    def _wait_g(b):
        pltpu.make_async_copy(y_hbm.at[ridx.at[0]], rows.at[b], gsem[b]).wait()

    def _wait_i(b):
        pltpu.make_async_copy(col_hbm.at[wid, 0, 0], cidx[b], isem[b]).wait()

    def _group(g, carry):
        for b in range(NBUF):
            j = g * NBUF + b
            _wait_g(b)
            _wait_i(b)
            pltpu.sync_copy(rows.at[b], acc.at[cidx[b]], add=True)
            pltpu.async_copy(col_hbm.at[wid, j + NBUF, 0], cidx[b], isem[b])
            pltpu.async_copy(y_hbm.at[ridx.at[j + NBUF]], rows.at[b], gsem[b])
        return carry

    NG = (NCHUNK - NBUF) // NBUF
    lax.fori_loop(0, NG, _group, 0)
    for j in range(NG * NBUF, NCHUNK):
        b = j % NBUF
        _wait_g(b)
        _wait_i(b)
        pltpu.sync_copy(rows.at[b], acc.at[cidx[b]], add=True)
        if j + NBUF < NCHUNK:
            pltpu.async_copy(col_hbm.at[wid, j + NBUF, 0], cidx[b], isem[b])
            pltpu.async_copy(y_hbm.at[ridx.at[j + NBUF]], rows.at[b], gsem[b])
    plsc.subcore_barrier()

    pltpu.sync_copy(acc.at[pl.ds(s * NPT, NPT)], out_hbm.at[c * NS + s])


def _sc_degree_body(col_hbm, ones_hbm, z_hbm, out_hbm, acc, cidx0, cidx1,
                    ones_b, i0, i1):
    c = lax.axis_index("c")
    s = lax.axis_index("s")
    wid = s * NC + c
    cidx = (cidx0, cidx1)
    isem = (i0, i1)

    pltpu.sync_copy(ones_hbm, ones_b)
    pltpu.sync_copy(z_hbm.at[s], acc.at[pl.ds(s * NPT, NPT)])
    for b in range(NBUF):
        pltpu.async_copy(col_hbm.at[wid, b, 0], cidx[b], isem[b])
    plsc.subcore_barrier()

    def _wait_i(b):
        pltpu.make_async_copy(col_hbm.at[wid, 0, 0], cidx[b], isem[b]).wait()

    def _group(g, carry):
        for b in range(NBUF):
            j = g * NBUF + b
            _wait_i(b)
            pltpu.sync_copy(ones_b, acc.at[cidx[b]], add=True)
            pltpu.async_copy(col_hbm.at[wid, j + NBUF, 0], cidx[b], isem[b])
        return carry

    NG = (NCHUNK - NBUF) // NBUF
    lax.fori_loop(0, NG, _group, 0)
    for j in range(NG * NBUF, NCHUNK):
        b = j % NBUF
        _wait_i(b)
        pltpu.sync_copy(ones_b, acc.at[cidx[b]], add=True)
        if j + NBUF < NCHUNK:
            pltpu.async_copy(col_hbm.at[wid, j + NBUF, 0], cidx[b], isem[b])
    plsc.subcore_barrier()

    pltpu.sync_copy(acc.at[pl.ds(s * NPT, NPT)], out_hbm.at[c * NS + s])


@functools.lru_cache(maxsize=None)
def _sc_kernels():
    mesh = plsc.VectorSubcoreMesh(core_axis_name="c", subcore_axis_name="s",
                                  num_cores=NC, num_subcores=NS)
    scatter = functools.partial(
        pl.kernel,
        out_type=jax.ShapeDtypeStruct((NC * NS, NPT, F), jnp.float32),
        mesh=mesh,
        scratch_types=[
            pltpu.VMEM_SHARED((N, F), jnp.float32),
            pltpu.VMEM((NCHUNK, K), jnp.int32),
            pltpu.VMEM((K,), jnp.int32),
            pltpu.VMEM((K,), jnp.int32),
            pltpu.VMEM((NBUF, K, F), jnp.float32),
            pltpu.SemaphoreType.DMA,
            pltpu.SemaphoreType.DMA,
            pltpu.SemaphoreType.DMA,
            pltpu.SemaphoreType.DMA,
        ],
    )(_sc_scatter_body)
    degree = functools.partial(
        pl.kernel,
        out_type=jax.ShapeDtypeStruct((NC * NS, NPT, F), jnp.float32),
        mesh=mesh,
        scratch_types=[
            pltpu.VMEM_SHARED((N, F), jnp.float32),
            pltpu.VMEM((K,), jnp.int32),
            pltpu.VMEM((K,), jnp.int32),
            pltpu.VMEM((K, F), jnp.float32),
            pltpu.SemaphoreType.DMA,
            pltpu.SemaphoreType.DMA,
        ],
    )(_sc_degree_body)
    return scatter, degree


def _tc_pre_body(p_ref, x_ref, w_ref, y_ref, dis_ref):
    deg = p_ref[0, :, 0:1] + p_ref[1, :, 0:1] + 1.0
    dis = lax.rsqrt(deg)
    dis_ref[...] = dis
    y_ref[...] = dis * jnp.dot(x_ref[...], w_ref[...],
                               preferred_element_type=jnp.float32)


def _tc_pre(p, x, w):
    return pl.pallas_call(
        _tc_pre_body,
        grid=(GRID,),
        in_specs=[
            pl.BlockSpec((NC, BN, F), lambda i: (0, i, 0)),
            pl.BlockSpec((BN, F), lambda i: (i, 0)),
            pl.BlockSpec((F, F), lambda i: (0, 0)),
        ],
        out_specs=[
            pl.BlockSpec((BN, F), lambda i: (i, 0)),
            pl.BlockSpec((BN, 1), lambda i: (i, 0)),
        ],
        out_shape=[
            jax.ShapeDtypeStruct((N, F), jnp.float32),
            jax.ShapeDtypeStruct((N, 1), jnp.float32),
        ],
    )(p, x, w)


def _tc_mid_body(s_ref, y_ref, dis_ref, b_ref, w_ref, yo_ref):
    t = s_ref[0] + s_ref[1] + y_ref[...]
    h = jnp.maximum(dis_ref[...] * t + b_ref[...], 0.0)
    yo_ref[...] = dis_ref[...] * jnp.dot(h, w_ref[...],
                                         preferred_element_type=jnp.float32)


def _tc_mid(s, y, dis, b, w):
    return pl.pallas_call(
        _tc_mid_body,
        grid=(GRID,),
        in_specs=[
            pl.BlockSpec((NC, BN, F), lambda i: (0, i, 0)),
            pl.BlockSpec((BN, F), lambda i: (i, 0)),
            pl.BlockSpec((BN, 1), lambda i: (i, 0)),
            pl.BlockSpec((1, F), lambda i: (0, 0)),
            pl.BlockSpec((F, F), lambda i: (0, 0)),
        ],
        out_specs=pl.BlockSpec((BN, F), lambda i: (i, 0)),
        out_shape=jax.ShapeDtypeStruct((N, F), jnp.float32),
    )(s, y, dis, b, w)


def _tc_fin_body(s_ref, y_ref, dis_ref, b_ref, seg_ref, wh_ref, bh_ref,
                 out_ref, sums, cnts):
    i = pl.program_id(0)

    @pl.when(i == 0)
    def _init():
        sums[...] = jnp.zeros_like(sums)
        cnts[...] = jnp.zeros_like(cnts)

    t = s_ref[0] + s_ref[1] + y_ref[...]
    h = dis_ref[...] * t + b_ref[...]
    onehot = (seg_ref[...] == lax.broadcasted_iota(jnp.int32, (1, G), 1)
              ).astype(jnp.float32)
    dn = (((0,), (0,)), ((), ()))
    sums[...] += lax.dot_general(onehot, h, dn,
                                 preferred_element_type=jnp.float32)
    cnts[...] += lax.dot_general(onehot, jnp.ones_like(h), dn,
                                 preferred_element_type=jnp.float32)

    @pl.when(i == GRID - 1)
    def _fin():
        pooled = sums[...] / jnp.maximum(cnts[...], 1.0)
        out_ref[...] = jnp.dot(pooled, wh_ref[...],
                               preferred_element_type=jnp.float32) + bh_ref[...]


def _tc_fin(s, y, dis, b, seg, wh, bh):
    return pl.pallas_call(
        _tc_fin_body,
        grid=(GRID,),
        in_specs=[
            pl.BlockSpec((NC, BN, F), lambda i: (0, i, 0)),
            pl.BlockSpec((BN, F), lambda i: (i, 0)),
            pl.BlockSpec((BN, 1), lambda i: (i, 0)),
            pl.BlockSpec((1, F), lambda i: (0, 0)),
            pl.BlockSpec((BN, 1), lambda i: (i, 0)),
            pl.BlockSpec((F, 1), lambda i: (0, 0)),
            pl.BlockSpec((1, 1), lambda i: (0, 0)),
        ],
        out_specs=pl.BlockSpec((G, 1), lambda i: (0, 0)),
        out_shape=jax.ShapeDtypeStruct((G, 1), jnp.float32),
        scratch_shapes=[
            pltpu.VMEM((G, F), jnp.float32),
            pltpu.VMEM((G, F), jnp.float32),
        ],
    )(s, y, dis, b, seg, wh, bh)


def kernel(x, edge_index, batch, W1, b1, W2, b2, W3, b3, W4, b4, Wh, bh):
    row = edge_index[0].astype(jnp.int32).reshape(NW, NCHUNK, K)
    col = edge_index[1].astype(jnp.int32).reshape(NW, NCHUNK, 1, K)
    seg = batch.astype(jnp.int32).reshape(N, 1)
    zf = jnp.zeros((NS, NPT, F), jnp.float32)
    ones_kf = jnp.ones((K, F), jnp.float32)
    _sc_scatter, _sc_degree = _sc_kernels()

    p = _sc_degree(col, ones_kf, zf).reshape(NC, N, F)
    y, dis = _tc_pre(p, x, W1)

    s = _sc_scatter(y, row, col, zf).reshape(NC, N, F)
    y = _tc_mid(s, y, dis, b1.reshape(1, F), W2)
    s = _sc_scatter(y, row, col, zf).reshape(NC, N, F)
    y = _tc_mid(s, y, dis, b2.reshape(1, F), W3)
    s = _sc_scatter(y, row, col, zf).reshape(NC, N, F)
    y = _tc_mid(s, y, dis, b3.reshape(1, F), W4)
    s = _sc_scatter(y, row, col, zf).reshape(NC, N, F)
    return _tc_fin(s, y, dis, b4.reshape(1, F), seg, Wh, bh.reshape(1, 1))
